# baseline (speedup 1.0000x reference)
import functools

import jax
import jax.numpy as jnp
from jax import lax
from jax.experimental import pallas as pl
from jax.experimental.pallas import tpu as pltpu
from jax.experimental.pallas import tpu_sc as plsc

N_NODES = 10000
N_EDGES = 320000
D = 128

NC = 2
NS = 16
NW = NC * NS

CSZ = 64
CH = 160
IB = 8
NBLK = CH // IB
NBUF = 4
EPT = CH * CSZ
E_PAD = NW * EPT

N_PAD = 10240
RPT = N_PAD // NS
RCH = RPT // CSZ
PAD_DST = N_NODES + 8

_sc_mesh = plsc.VectorSubcoreMesh(core_axis_name="c", subcore_axis_name="s")


@functools.partial(
    pl.kernel,
    out_type=jax.ShapeDtypeStruct((NC, N_PAD, D), jnp.float32),
    mesh=_sc_mesh,
    scratch_types=[
        pltpu.VMEM_SHARED((N_PAD, D), jnp.float32),
        pltpu.VMEM((IB, CSZ), jnp.int32),
        pltpu.VMEM((IB, CSZ), jnp.int32),
        [pltpu.VMEM((CSZ, D), jnp.float32)] * NBUF,
        [pltpu.SemaphoreType.DMA] * NBUF,
        [pltpu.SemaphoreType.DMA] * NBUF,
    ],
)
def _sc_aggregate(x_hbm, src_hbm, dst_hbm, out_hbm,
                  acc, src_v, dst_v, bufs, gsems, ssems):
    cid = lax.axis_index("c")
    sid = lax.axis_index("s")
    gid = cid * NS + sid
    r0 = sid * RPT

    for k in range(RCH):
        rows = pl.ds(r0 + k * CSZ, CSZ)
        pltpu.sync_copy(x_hbm.at[rows], bufs[0])
        pltpu.sync_copy(bufs[0], acc.at[rows])
    plsc.subcore_barrier()

    def gath(j, s):
        return pltpu.async_copy(x_hbm.at[src_v.at[j]], bufs[s], gsems[s])

    def scat(j, s):
        return pltpu.async_copy(bufs[s], acc.at[dst_v.at[j]], ssems[s],
                                add=True)

    def wait_scat(j, s):
        pltpu.make_async_copy(bufs[s], acc.at[dst_v.at[j]], ssems[s]).wait()

    def body(ib, carry):
        @pl.when(ib > 0)
        def _drain_prev():
            for s in range(NBUF):
                wait_scat(IB - NBUF + s, s)

        pltpu.sync_copy(src_hbm.at[gid, pl.ds(ib * IB, IB)], src_v)
        pltpu.sync_copy(dst_hbm.at[gid, pl.ds(ib * IB, IB)], dst_v)

        g = [None] * NBUF
        for s in range(NBUF):
            g[s] = gath(s, s)
        for j in range(IB):
            s = j % NBUF
            g[s].wait()
            scat(j, s)
            k = j + NBUF - 1
            if NBUF <= k < IB:
                s2 = k % NBUF
                wait_scat(j - 1, s2)
                g[s2] = gath(k, s2)
        return carry

    lax.fori_loop(0, NBLK, body, 0)
    for s in range(NBUF):
        wait_scat(IB - NBUF + s, s)
    plsc.subcore_barrier()

    for k in range(RCH):
        rows = pl.ds(r0 + k * CSZ, CSZ)
        pltpu.sync_copy(acc.at[rows], bufs[0])
        pltpu.sync_copy(bufs[0], out_hbm.at[cid, rows])


_ROWS_BLK = 1024


def _mlp_body(final_relu, x_ref, p0_ref, p1_ref, wa_ref, ba_ref, wb_ref,
              bb_ref, o_ref):
    h = p0_ref[...] + p1_ref[...] - x_ref[...]
    h = jnp.dot(h, wa_ref[...], preferred_element_type=jnp.float32)
    h = jnp.maximum(h + ba_ref[...], 0.0)
    o = jnp.dot(h, wb_ref[...], preferred_element_type=jnp.float32)
    o = o + bb_ref[...]
    if final_relu:
        o = jnp.maximum(o, 0.0)
    o_ref[...] = o


def _mlp(x_pad, p0, p1, wa, ba, wb, bb, final_relu):
    row_spec = pl.BlockSpec((_ROWS_BLK, D), lambda i: (i, 0))
    full_spec = pl.BlockSpec((D, D), lambda i: (0, 0))
    bias_spec = pl.BlockSpec((1, D), lambda i: (0, 0))
    return pl.pallas_call(
        functools.partial(_mlp_body, final_relu),
        grid=(N_PAD // _ROWS_BLK,),
        in_specs=[row_spec, row_spec, row_spec,
                  full_spec, bias_spec, full_spec, bias_spec],
        out_specs=row_spec,
        out_shape=jax.ShapeDtypeStruct((N_PAD, D), jnp.float32),
    )(x_pad, p0, p1, wa, ba.reshape(1, D), wb, bb.reshape(1, D))


def kernel(x, edge_index, W1a, b1a, W1b, b1b, W2a, b2a, W2b, b2b):
    src = edge_index[0].astype(jnp.int32)
    dst = edge_index[1].astype(jnp.int32)
    pad_e = E_PAD - N_EDGES
    src_r = jnp.concatenate([src, jnp.zeros((pad_e,), jnp.int32)])
    src_r = src_r.reshape(NW, CH, CSZ)
    dst_r = jnp.concatenate([dst, jnp.full((pad_e,), PAD_DST, jnp.int32)])
    dst_r = dst_r.reshape(NW, CH, CSZ)
    x_pad = jnp.concatenate(
        [x, jnp.zeros((N_PAD - N_NODES, D), jnp.float32)])

    parts1 = _sc_aggregate(x_pad, src_r, dst_r)
    h1 = _mlp(x_pad, parts1[0], parts1[1], W1a, b1a, W1b, b1b,
              final_relu=True)
    parts2 = _sc_aggregate(h1, src_r, dst_r)
    out = _mlp(h1, parts2[0], parts2[1], W2a, b2a, W2b, b2b,
               final_relu=False)
    return out[:N_NODES]


# speedup vs baseline: 1.1628x; 1.1628x over previous
import functools

import jax
import jax.numpy as jnp
from jax import lax
from jax.experimental import pallas as pl
from jax.experimental.pallas import tpu as pltpu
from jax.experimental.pallas import tpu_sc as plsc

N_NODES = 10000
N_EDGES = 320000
D = 128

NC = 2
NS = 16
NW = NC * NS

CSZ = 128
CH = 80
IB = 8
NBLK = CH // IB
NBUF = 2
EPT = CH * CSZ
E_PAD = NW * EPT

N_PAD = 10240
RPT = N_PAD // NS
RCH = RPT // CSZ
PAD_DST = N_NODES + 8

_sc_mesh = plsc.VectorSubcoreMesh(core_axis_name="c", subcore_axis_name="s")


@functools.partial(
    pl.kernel,
    out_type=jax.ShapeDtypeStruct((NC, N_PAD, D), jnp.float32),
    mesh=_sc_mesh,
    scratch_types=[
        pltpu.VMEM_SHARED((N_PAD, D), jnp.float32),
        pltpu.VMEM((2 * IB, CSZ), jnp.int32),
        [pltpu.VMEM((CSZ, D), jnp.float32)] * NBUF,
        [pltpu.SemaphoreType.DMA] * NBUF,
        [pltpu.SemaphoreType.DMA] * NBUF,
    ],
)
def _sc_aggregate(x_hbm, idx_hbm, out_hbm,
                  acc, idx_v, bufs, gsems, ssems):
    cid = lax.axis_index("c")
    sid = lax.axis_index("s")
    gid = cid * NS + sid
    r0 = sid * RPT

    for k in range(RCH):
        rows = pl.ds(r0 + k * CSZ, CSZ)
        pltpu.sync_copy(x_hbm.at[rows], bufs[0])
        pltpu.sync_copy(bufs[0], acc.at[rows])
    plsc.subcore_barrier()

    def gath(j, s):
        return pltpu.async_copy(x_hbm.at[idx_v.at[j]], bufs[s], gsems[s])

    def scat(j, s):
        return pltpu.async_copy(bufs[s], acc.at[idx_v.at[IB + j]], ssems[s],
                                add=True)

    def wait_scat(j, s):
        pltpu.make_async_copy(bufs[s], acc.at[idx_v.at[IB + j]],
                              ssems[s]).wait()

    def body(ib, carry):
        pltpu.sync_copy(idx_hbm.at[gid, ib], idx_v)
        g0 = gath(0, 0)
        g1 = gath(1, 1)
        for p in range(IB // 2):
            a = 2 * p
            b = a + 1
            g0.wait()
            scat(a, 0)
            g1.wait()
            scat(b, 1)
            if p < IB // 2 - 1:
                wait_scat(a, 0)
                g0 = gath(a + 2, 0)
                wait_scat(b, 1)
                g1 = gath(b + 2, 1)
            else:
                wait_scat(a, 0)
                wait_scat(b, 1)
        return carry

    lax.fori_loop(0, NBLK, body, 0)
    plsc.subcore_barrier()

    for k in range(RCH):
        rows = pl.ds(r0 + k * CSZ, CSZ)
        pltpu.sync_copy(acc.at[rows], bufs[0])
        pltpu.sync_copy(bufs[0], out_hbm.at[cid, rows])


_ROWS_BLK = 1024


def _mlp_body(final_relu, x_ref, p0_ref, p1_ref, wa_ref, ba_ref, wb_ref,
              bb_ref, o_ref):
    h = p0_ref[...] + p1_ref[...] - x_ref[...]
    h = jnp.dot(h, wa_ref[...], preferred_element_type=jnp.float32)
    h = jnp.maximum(h + ba_ref[...], 0.0)
    o = jnp.dot(h, wb_ref[...], preferred_element_type=jnp.float32)
    o = o + bb_ref[...]
    if final_relu:
        o = jnp.maximum(o, 0.0)
    o_ref[...] = o


def _mlp(x_pad, p0, p1, wa, ba, wb, bb, final_relu):
    row_spec = pl.BlockSpec((_ROWS_BLK, D), lambda i: (i, 0))
    full_spec = pl.BlockSpec((D, D), lambda i: (0, 0))
    bias_spec = pl.BlockSpec((1, D), lambda i: (0, 0))
    return pl.pallas_call(
        functools.partial(_mlp_body, final_relu),
        grid=(N_PAD // _ROWS_BLK,),
        in_specs=[row_spec, row_spec, row_spec,
                  full_spec, bias_spec, full_spec, bias_spec],
        out_specs=row_spec,
        out_shape=jax.ShapeDtypeStruct((N_PAD, D), jnp.float32),
    )(x_pad, p0, p1, wa, ba.reshape(1, D), wb, bb.reshape(1, D))


def kernel(x, edge_index, W1a, b1a, W1b, b1b, W2a, b2a, W2b, b2b):
    src = edge_index[0].astype(jnp.int32)
    dst = edge_index[1].astype(jnp.int32)
    pad_e = E_PAD - N_EDGES
    src_r = jnp.concatenate([src, jnp.zeros((pad_e,), jnp.int32)])
    src_r = src_r.reshape(NW, NBLK, IB, CSZ)
    dst_r = jnp.concatenate([dst, jnp.full((pad_e,), PAD_DST, jnp.int32)])
    dst_r = dst_r.reshape(NW, NBLK, IB, CSZ)
    idx_comb = jnp.concatenate([src_r, dst_r], axis=2)
    x_pad = jnp.concatenate(
        [x, jnp.zeros((N_PAD - N_NODES, D), jnp.float32)])

    parts1 = _sc_aggregate(x_pad, idx_comb)
    h1 = _mlp(x_pad, parts1[0], parts1[1], W1a, b1a, W1b, b1b,
              final_relu=True)
    parts2 = _sc_aggregate(h1, idx_comb)
    out = _mlp(h1, parts2[0], parts2[1], W2a, b2a, W2b, b2b,
               final_relu=False)
    return out[:N_NODES]


# speedup vs baseline: 2.4195x; 2.0808x over previous
import functools

import jax
import jax.numpy as jnp
from jax import lax
from jax.experimental import pallas as pl
from jax.experimental.pallas import tpu as pltpu
from jax.experimental.pallas import tpu_sc as plsc

N_NODES = 10000
N_EDGES = 320000
D = 128

NC = 2
NS = 16
DH = D // NC

CSZ = 128
CH = 160
IB = 8
NBLK = CH // IB
NBUF = 2
EPT = CH * CSZ
E_PAD = NS * EPT

N_PAD = 10240
RPT = N_PAD // NS
RCH = RPT // CSZ
PAD_DST = N_NODES + 8

_sc_mesh = plsc.VectorSubcoreMesh(core_axis_name="c", subcore_axis_name="s")


@functools.partial(
    pl.kernel,
    out_type=jax.ShapeDtypeStruct((NC, N_PAD, DH), jnp.float32),
    mesh=_sc_mesh,
    scratch_types=[
        pltpu.VMEM_SHARED((N_PAD, DH), jnp.float32),
        pltpu.VMEM_SHARED((N_PAD, DH), jnp.float32),
        pltpu.VMEM((2 * IB, CSZ), jnp.int32),
        [pltpu.VMEM((CSZ, DH), jnp.float32)] * NBUF,
        [pltpu.SemaphoreType.DMA] * NBUF,
        [pltpu.SemaphoreType.DMA] * NBUF,
    ],
)
def _sc_aggregate(x_hbm, idx_hbm, out_hbm,
                  tbl, acc, idx_v, bufs, gsems, ssems):
    cid = lax.axis_index("c")
    sid = lax.axis_index("s")
    r0 = sid * RPT

    for k in range(RCH):
        rows = pl.ds(r0 + k * CSZ, CSZ)
        pltpu.sync_copy(x_hbm.at[cid, rows], bufs[0])
        pltpu.sync_copy(bufs[0], tbl.at[rows])
        pltpu.sync_copy(bufs[0], acc.at[rows])
    plsc.subcore_barrier()

    def gath(j, s):
        return pltpu.async_copy(tbl.at[idx_v.at[j]], bufs[s], gsems[s])

    def scat(j, s):
        return pltpu.async_copy(bufs[s], acc.at[idx_v.at[IB + j]], ssems[s],
                                add=True)

    def wait_scat(j, s):
        pltpu.make_async_copy(bufs[s], acc.at[idx_v.at[IB + j]],
                              ssems[s]).wait()

    def body(ib, carry):
        pltpu.sync_copy(idx_hbm.at[sid, ib], idx_v)
        g0 = gath(0, 0)
        g1 = gath(1, 1)
        for p in range(IB // 2):
            a = 2 * p
            b = a + 1
            g0.wait()
            scat(a, 0)
            g1.wait()
            scat(b, 1)
            if p < IB // 2 - 1:
                wait_scat(a, 0)
                g0 = gath(a + 2, 0)
                wait_scat(b, 1)
                g1 = gath(b + 2, 1)
            else:
                wait_scat(a, 0)
                wait_scat(b, 1)
        return carry

    lax.fori_loop(0, NBLK, body, 0)
    plsc.subcore_barrier()

    for k in range(RCH):
        rows = pl.ds(r0 + k * CSZ, CSZ)
        pltpu.sync_copy(acc.at[rows], bufs[0])
        pltpu.sync_copy(bufs[0], out_hbm.at[cid, rows])


_ROWS_BLK = 1024


def _mlp_body(split_out, plo_ref, phi_ref, wa_ref, ba_ref, wb_ref,
              bb_ref, o_ref):
    h = jnp.concatenate([plo_ref[...], phi_ref[...]], axis=-1)
    h = jnp.dot(h, wa_ref[...], preferred_element_type=jnp.float32)
    h = jnp.maximum(h + ba_ref[...], 0.0)
    o = jnp.dot(h, wb_ref[...], preferred_element_type=jnp.float32)
    o = o + bb_ref[...]
    if split_out:
        o = jnp.maximum(o, 0.0)
        o_ref[0] = o[:, :DH]
        o_ref[1] = o[:, DH:]
    else:
        o_ref[...] = o


def _mlp(plo, phi, wa, ba, wb, bb, split_out):
    half_spec = pl.BlockSpec((_ROWS_BLK, DH), lambda i: (i, 0))
    full_spec = pl.BlockSpec((D, D), lambda i: (0, 0))
    bias_spec = pl.BlockSpec((1, D), lambda i: (0, 0))
    if split_out:
        out_spec = pl.BlockSpec((NC, _ROWS_BLK, DH), lambda i: (0, i, 0))
        out_shape = jax.ShapeDtypeStruct((NC, N_PAD, DH), jnp.float32)
    else:
        out_spec = pl.BlockSpec((_ROWS_BLK, D), lambda i: (i, 0))
        out_shape = jax.ShapeDtypeStruct((N_PAD, D), jnp.float32)
    return pl.pallas_call(
        functools.partial(_mlp_body, split_out),
        grid=(N_PAD // _ROWS_BLK,),
        in_specs=[half_spec, half_spec,
                  full_spec, bias_spec, full_spec, bias_spec],
        out_specs=out_spec,
        out_shape=out_shape,
    )(plo, phi, wa, ba.reshape(1, D), wb, bb.reshape(1, D))


def kernel(x, edge_index, W1a, b1a, W1b, b1b, W2a, b2a, W2b, b2b):
    src = edge_index[0].astype(jnp.int32)
    dst = edge_index[1].astype(jnp.int32)
    pad_e = E_PAD - N_EDGES
    src_r = jnp.concatenate([src, jnp.zeros((pad_e,), jnp.int32)])
    src_r = src_r.reshape(NS, NBLK, IB, CSZ)
    dst_r = jnp.concatenate([dst, jnp.full((pad_e,), PAD_DST, jnp.int32)])
    dst_r = dst_r.reshape(NS, NBLK, IB, CSZ)
    idx_comb = jnp.concatenate([src_r, dst_r], axis=2)
    x_pad = jnp.concatenate(
        [x, jnp.zeros((N_PAD - N_NODES, D), jnp.float32)])
    x2 = jnp.stack([x_pad[:, :DH], x_pad[:, DH:]])

    parts1 = _sc_aggregate(x2, idx_comb)
    h1_2 = _mlp(parts1[0], parts1[1], W1a, b1a, W1b, b1b, split_out=True)
    parts2 = _sc_aggregate(h1_2, idx_comb)
    out = _mlp(parts2[0], parts2[1], W2a, b2a, W2b, b2b, split_out=False)
    return out[:N_NODES]


# speedup vs baseline: 2.9113x; 1.2032x over previous
import functools

import jax
import jax.numpy as jnp
from jax import lax
from jax.experimental import pallas as pl
from jax.experimental.pallas import tpu as pltpu
from jax.experimental.pallas import tpu_sc as plsc

N_NODES = 10000
N_EDGES = 320000
D = 128

NC = 2
NS = 16
DH = D // NC

CSZ = 128
CH = 160
IB = 8
NBLK = CH // IB
NBUF = 2
EPT = CH * CSZ
E_PAD = NS * EPT

N_PAD = 10240
RPT = N_PAD // NS
RCH = RPT // CSZ
PAD_DST = N_NODES + 8

_sc_mesh = plsc.VectorSubcoreMesh(core_axis_name="c", subcore_axis_name="s")


@functools.partial(
    pl.kernel,
    out_type=jax.ShapeDtypeStruct((NC, N_PAD, DH), jnp.float32),
    mesh=_sc_mesh,
    scratch_types=[
        pltpu.VMEM_SHARED((N_PAD, DH), jnp.float32),
        pltpu.VMEM_SHARED((N_PAD, DH), jnp.float32),
        [pltpu.VMEM((2 * IB, CSZ), jnp.int32)] * 2,
        [pltpu.VMEM((CSZ, DH), jnp.float32)] * NBUF,
        [pltpu.SemaphoreType.DMA] * NBUF,
        [pltpu.SemaphoreType.DMA] * NBUF,
        [pltpu.SemaphoreType.DMA] * 2,
    ],
)
def _sc_aggregate(x_hbm, idx_hbm, out_hbm,
                  tbl, acc, idxs, bufs, gsems, ssems, isems):
    cid = lax.axis_index("c")
    sid = lax.axis_index("s")
    r0 = sid * RPT

    rows = pl.ds(r0, RPT)
    pltpu.sync_copy(x_hbm.at[cid, rows], tbl.at[rows])
    pltpu.sync_copy(x_hbm.at[cid, rows], acc.at[rows])
    plsc.subcore_barrier()

    def gath(iv, j, s):
        pltpu.async_copy(tbl.at[iv.at[j]], bufs[s], gsems[s])

    def wait_gath(iv, s):
        pltpu.make_async_copy(tbl.at[iv.at[0]], bufs[s], gsems[s]).wait()

    def scat(iv, j, s):
        pltpu.async_copy(bufs[s], acc.at[iv.at[IB + j]], ssems[s], add=True)

    def wait_scat(iv, s):
        pltpu.make_async_copy(bufs[s], acc.at[iv.at[IB]], ssems[s]).wait()

    def fetch_idx(blk, which):
        pltpu.async_copy(idx_hbm.at[sid, blk], idxs[which], isems[which])

    def wait_idx(which):
        pltpu.make_async_copy(idx_hbm.at[sid, 0], idxs[which],
                              isems[which]).wait()

    def process_block(cur, nxt, nxt_ready, chain_wait):
        for p in range(IB // 2):
            a = 2 * p
            b = a + 1
            wait_gath(cur, 0)
            scat(cur, a, 0)
            wait_gath(cur, 1)
            scat(cur, b, 1)
            if p < IB // 2 - 1:
                wait_scat(cur, 0)
                gath(cur, a + 2, 0)
                wait_scat(cur, 1)
                gath(cur, b + 2, 1)
            else:
                @pl.when(nxt_ready)
                def _chain():
                    chain_wait()
                    wait_scat(cur, 0)
                    gath(nxt, 0, 0)
                    wait_scat(cur, 1)
                    gath(nxt, 1, 1)

                @pl.when(jnp.logical_not(nxt_ready))
                def _drain():
                    wait_scat(cur, 0)
                    wait_scat(cur, 1)

    pltpu.sync_copy(idx_hbm.at[sid, 0], idxs[0])
    fetch_idx(1, 1)
    gath(idxs[0], 0, 0)
    gath(idxs[0], 1, 1)

    def body(i, carry):
        blk = 2 * i
        process_block(idxs[0], idxs[1], jnp.bool_(True),
                      lambda: wait_idx(1))
        @pl.when(blk + 2 < NBLK)
        def _pf0():
            fetch_idx(blk + 2, 0)
        process_block(idxs[1], idxs[0], blk + 2 < NBLK,
                      lambda: wait_idx(0))
        @pl.when(blk + 3 < NBLK)
        def _pf1():
            fetch_idx(blk + 3, 1)
        return carry

    lax.fori_loop(0, NBLK // 2, body, 0)
    plsc.subcore_barrier()

    pltpu.sync_copy(acc.at[rows], out_hbm.at[cid, rows])


_ROWS_BLK = 1024


def _mlp_body(split_out, plo_ref, phi_ref, wa_ref, ba_ref, wb_ref,
              bb_ref, o_ref):
    h = jnp.concatenate([plo_ref[...], phi_ref[...]], axis=-1)
    h = jnp.dot(h, wa_ref[...], preferred_element_type=jnp.float32)
    h = jnp.maximum(h + ba_ref[...], 0.0)
    o = jnp.dot(h, wb_ref[...], preferred_element_type=jnp.float32)
    o = o + bb_ref[...]
    if split_out:
        o = jnp.maximum(o, 0.0)
        o_ref[0] = o[:, :DH]
        o_ref[1] = o[:, DH:]
    else:
        o_ref[...] = o


def _mlp(plo, phi, wa, ba, wb, bb, split_out):
    half_spec = pl.BlockSpec((_ROWS_BLK, DH), lambda i: (i, 0))
    full_spec = pl.BlockSpec((D, D), lambda i: (0, 0))
    bias_spec = pl.BlockSpec((1, D), lambda i: (0, 0))
    if split_out:
        out_spec = pl.BlockSpec((NC, _ROWS_BLK, DH), lambda i: (0, i, 0))
        out_shape = jax.ShapeDtypeStruct((NC, N_PAD, DH), jnp.float32)
    else:
        out_spec = pl.BlockSpec((_ROWS_BLK, D), lambda i: (i, 0))
        out_shape = jax.ShapeDtypeStruct((N_PAD, D), jnp.float32)
    return pl.pallas_call(
        functools.partial(_mlp_body, split_out),
        grid=(N_PAD // _ROWS_BLK,),
        in_specs=[half_spec, half_spec,
                  full_spec, bias_spec, full_spec, bias_spec],
        out_specs=out_spec,
        out_shape=out_shape,
    )(plo, phi, wa, ba.reshape(1, D), wb, bb.reshape(1, D))


def kernel(x, edge_index, W1a, b1a, W1b, b1b, W2a, b2a, W2b, b2b):
    src = edge_index[0].astype(jnp.int32)
    dst = edge_index[1].astype(jnp.int32)
    pad_e = E_PAD - N_EDGES
    src_r = jnp.concatenate([src, jnp.zeros((pad_e,), jnp.int32)])
    src_r = src_r.reshape(NS, NBLK, IB, CSZ)
    dst_r = jnp.concatenate([dst, jnp.full((pad_e,), PAD_DST, jnp.int32)])
    dst_r = dst_r.reshape(NS, NBLK, IB, CSZ)
    idx_comb = jnp.concatenate([src_r, dst_r], axis=2)
    x_pad = jnp.concatenate(
        [x, jnp.zeros((N_PAD - N_NODES, D), jnp.float32)])
    x2 = jnp.stack([x_pad[:, :DH], x_pad[:, DH:]])

    parts1 = _sc_aggregate(x2, idx_comb)
    h1_2 = _mlp(parts1[0], parts1[1], W1a, b1a, W1b, b1b, split_out=True)
    parts2 = _sc_aggregate(h1_2, idx_comb)
    out = _mlp(parts2[0], parts2[1], W2a, b2a, W2b, b2b, split_out=False)
    return out[:N_NODES]


# speedup vs baseline: 2.9428x; 1.0108x over previous
import functools

import jax
import jax.numpy as jnp
from jax import lax
from jax.experimental import pallas as pl
from jax.experimental.pallas import tpu as pltpu
from jax.experimental.pallas import tpu_sc as plsc

N_NODES = 10000
N_EDGES = 320000
D = 128

NC = 2
NS = 16
DH = D // NC

CSZ = 128
CH = 160
IB = 8
NBLK = CH // IB
NBUF = 2
EPT = CH * CSZ
E_PAD = NS * EPT

N_PAD = 10240
RPT = N_PAD // NS
RCH = RPT // CSZ
PAD_DST = N_NODES + 8

_sc_mesh = plsc.VectorSubcoreMesh(core_axis_name="c", subcore_axis_name="s")


@functools.partial(
    pl.kernel,
    out_type=jax.ShapeDtypeStruct((NC, N_PAD, DH), jnp.float32),
    mesh=_sc_mesh,
    scratch_types=[
        pltpu.VMEM_SHARED((N_PAD, DH), jnp.float32),
        pltpu.VMEM_SHARED((N_PAD, DH), jnp.float32),
        [pltpu.VMEM((2 * IB, CSZ), jnp.int32)] * 2,
        [pltpu.VMEM((CSZ, DH), jnp.float32)] * NBUF,
        [pltpu.SemaphoreType.DMA] * NBUF,
        [pltpu.SemaphoreType.DMA] * NBUF,
        [pltpu.SemaphoreType.DMA] * 2,
    ],
)
def _sc_aggregate(x_hbm, idx_hbm, out_hbm,
                  tbl, acc, idxs, bufs, gsems, ssems, isems):
    cid = lax.axis_index("c")
    sid = lax.axis_index("s")
    r0 = sid * RPT

    rows = pl.ds(r0, RPT)
    pltpu.sync_copy(x_hbm.at[cid, rows], tbl.at[rows])
    pltpu.sync_copy(x_hbm.at[cid, rows], acc.at[rows])
    plsc.subcore_barrier()

    def gath(iv, j, s):
        pltpu.async_copy(tbl.at[iv.at[j]], bufs[s], gsems[s])

    def wait_gath(iv, s):
        pltpu.make_async_copy(tbl.at[iv.at[0]], bufs[s], gsems[s]).wait()

    def scat(iv, j, s):
        pltpu.async_copy(bufs[s], acc.at[iv.at[IB + j]], ssems[s], add=True)

    def wait_scat(iv, s):
        pltpu.make_async_copy(bufs[s], acc.at[iv.at[IB]], ssems[s]).wait()

    def fetch_idx(blk, which):
        pltpu.async_copy(idx_hbm.at[sid, blk], idxs[which], isems[which])

    def wait_idx(which):
        pltpu.make_async_copy(idx_hbm.at[sid, 0], idxs[which],
                              isems[which]).wait()

    def process_block(cur, nxt, nxt_ready, chain_wait):
        for p in range(IB // 2):
            a = 2 * p
            b = a + 1
            wait_gath(cur, 0)
            scat(cur, a, 0)
            wait_gath(cur, 1)
            scat(cur, b, 1)
            if p < IB // 2 - 1:
                wait_scat(cur, 0)
                gath(cur, a + 2, 0)
                wait_scat(cur, 1)
                gath(cur, b + 2, 1)
            else:
                @pl.when(nxt_ready)
                def _chain():
                    chain_wait()
                    wait_scat(cur, 0)
                    gath(nxt, 0, 0)
                    wait_scat(cur, 1)
                    gath(nxt, 1, 1)

                @pl.when(jnp.logical_not(nxt_ready))
                def _drain():
                    wait_scat(cur, 0)
                    wait_scat(cur, 1)

    pltpu.sync_copy(idx_hbm.at[sid, 0], idxs[0])
    fetch_idx(1, 1)
    gath(idxs[0], 0, 0)
    gath(idxs[0], 1, 1)

    def body(i, carry):
        blk = 2 * i
        process_block(idxs[0], idxs[1], jnp.bool_(True),
                      lambda: wait_idx(1))
        @pl.when(blk + 2 < NBLK)
        def _pf0():
            fetch_idx(blk + 2, 0)
        process_block(idxs[1], idxs[0], blk + 2 < NBLK,
                      lambda: wait_idx(0))
        @pl.when(blk + 3 < NBLK)
        def _pf1():
            fetch_idx(blk + 3, 1)
        return carry

    lax.fori_loop(0, NBLK // 2, body, 0)
    plsc.subcore_barrier()

    pltpu.sync_copy(acc.at[rows], out_hbm.at[cid, rows])


_ROWS_BLK = 2048


def _mlp_body(split_out, plo_ref, phi_ref, wa_ref, ba_ref, wb_ref,
              bb_ref, o_ref):
    h = jnp.concatenate([plo_ref[...], phi_ref[...]], axis=-1)
    h = jnp.dot(h, wa_ref[...], preferred_element_type=jnp.float32)
    h = jnp.maximum(h + ba_ref[...], 0.0)
    o = jnp.dot(h, wb_ref[...], preferred_element_type=jnp.float32)
    o = o + bb_ref[...]
    if split_out:
        o = jnp.maximum(o, 0.0)
        o_ref[0] = o[:, :DH]
        o_ref[1] = o[:, DH:]
    else:
        o_ref[...] = o


def _mlp(plo, phi, wa, ba, wb, bb, split_out):
    half_spec = pl.BlockSpec((_ROWS_BLK, DH), lambda i: (i, 0))
    full_spec = pl.BlockSpec((D, D), lambda i: (0, 0))
    bias_spec = pl.BlockSpec((1, D), lambda i: (0, 0))
    if split_out:
        out_spec = pl.BlockSpec((NC, _ROWS_BLK, DH), lambda i: (0, i, 0))
        out_shape = jax.ShapeDtypeStruct((NC, N_PAD, DH), jnp.float32)
    else:
        out_spec = pl.BlockSpec((_ROWS_BLK, D), lambda i: (i, 0))
        out_shape = jax.ShapeDtypeStruct((N_PAD, D), jnp.float32)
    return pl.pallas_call(
        functools.partial(_mlp_body, split_out),
        grid=(N_PAD // _ROWS_BLK,),
        in_specs=[half_spec, half_spec,
                  full_spec, bias_spec, full_spec, bias_spec],
        out_specs=out_spec,
        out_shape=out_shape,
    )(plo, phi, wa, ba.reshape(1, D), wb, bb.reshape(1, D))


def kernel(x, edge_index, W1a, b1a, W1b, b1b, W2a, b2a, W2b, b2b):
    src = edge_index[0].astype(jnp.int32)
    dst = edge_index[1].astype(jnp.int32)
    pad_e = E_PAD - N_EDGES
    src_r = jnp.concatenate([src, jnp.zeros((pad_e,), jnp.int32)])
    src_r = src_r.reshape(NS, NBLK, IB, CSZ)
    dst_r = jnp.concatenate([dst, jnp.full((pad_e,), PAD_DST, jnp.int32)])
    dst_r = dst_r.reshape(NS, NBLK, IB, CSZ)
    idx_comb = jnp.concatenate([src_r, dst_r], axis=2)
    x_pad = jnp.concatenate(
        [x, jnp.zeros((N_PAD - N_NODES, D), jnp.float32)])
    x2 = jnp.stack([x_pad[:, :DH], x_pad[:, DH:]])

    parts1 = _sc_aggregate(x2, idx_comb)
    h1_2 = _mlp(parts1[0], parts1[1], W1a, b1a, W1b, b1b, split_out=True)
    parts2 = _sc_aggregate(h1_2, idx_comb)
    out = _mlp(parts2[0], parts2[1], W2a, b2a, W2b, b2b, split_out=False)
    return out[:N_NODES]


# speedup vs baseline: 2.9936x; 1.0173x over previous
import functools

import jax
import jax.numpy as jnp
from jax import lax
from jax.experimental import pallas as pl
from jax.experimental.pallas import tpu as pltpu
from jax.experimental.pallas import tpu_sc as plsc

N_NODES = 10000
N_EDGES = 320000
D = 128

NC = 2
NS = 16
DH = D // NC

CSZ = 128
CH = 160
IB = 8
NBLK = CH // IB
NBUF = 2
EPT = CH * CSZ
E_PAD = NS * EPT

N_PAD = 10240
RPT = N_PAD // NS
RCH = RPT // CSZ
PAD_DST = N_NODES + 8

_sc_mesh = plsc.VectorSubcoreMesh(core_axis_name="c", subcore_axis_name="s")


@functools.partial(
    pl.kernel,
    out_type=jax.ShapeDtypeStruct((NC, N_PAD, DH), jnp.float32),
    mesh=_sc_mesh,
    scratch_types=[
        pltpu.VMEM_SHARED((N_PAD, DH), jnp.float32),
        pltpu.VMEM_SHARED((N_PAD, DH), jnp.float32),
        [pltpu.VMEM((2 * IB, CSZ), jnp.int32)] * 2,
        [pltpu.VMEM((CSZ, DH), jnp.float32)] * NBUF,
        [pltpu.SemaphoreType.DMA] * NBUF,
        [pltpu.SemaphoreType.DMA] * NBUF,
        [pltpu.SemaphoreType.DMA] * 2,
    ],
)
def _sc_aggregate(x_hbm, idx_hbm, out_hbm,
                  tbl, acc, idxs, bufs, gsems, ssems, isems):
    cid = lax.axis_index("c")
    sid = lax.axis_index("s")
    r0 = sid * RPT

    rows = pl.ds(r0, RPT)
    pltpu.sync_copy(x_hbm.at[cid, rows], tbl.at[rows])
    pltpu.sync_copy(x_hbm.at[cid, rows], acc.at[rows])
    plsc.subcore_barrier()

    def gath(iv, j, s):
        pltpu.async_copy(tbl.at[iv.at[j]], bufs[s], gsems[s])

    def wait_gath(iv, s):
        pltpu.make_async_copy(tbl.at[iv.at[0]], bufs[s], gsems[s]).wait()

    def scat(iv, j, s):
        pltpu.async_copy(bufs[s], acc.at[iv.at[IB + j]], ssems[s], add=True)

    def wait_scat(iv, s):
        pltpu.make_async_copy(bufs[s], acc.at[iv.at[IB]], ssems[s]).wait()

    def fetch_idx(blk, which):
        pltpu.async_copy(idx_hbm.at[sid, blk], idxs[which], isems[which])

    def wait_idx(which):
        pltpu.make_async_copy(idx_hbm.at[sid, 0], idxs[which],
                              isems[which]).wait()

    def process_block(cur, nxt, nxt_ready, chain_wait):
        for p in range(IB // 2):
            a = 2 * p
            b = a + 1
            wait_gath(cur, 0)
            scat(cur, a, 0)
            wait_gath(cur, 1)
            scat(cur, b, 1)
            if p < IB // 2 - 1:
                wait_scat(cur, 0)
                gath(cur, a + 2, 0)
                wait_scat(cur, 1)
                gath(cur, b + 2, 1)
            else:
                @pl.when(nxt_ready)
                def _chain():
                    chain_wait()
                    wait_scat(cur, 0)
                    gath(nxt, 0, 0)
                    wait_scat(cur, 1)
                    gath(nxt, 1, 1)

                @pl.when(jnp.logical_not(nxt_ready))
                def _drain():
                    wait_scat(cur, 0)
                    wait_scat(cur, 1)

    pltpu.sync_copy(idx_hbm.at[sid, 0], idxs[0])
    fetch_idx(1, 1)
    gath(idxs[0], 0, 0)
    gath(idxs[0], 1, 1)

    def body(i, carry):
        blk = 2 * i
        process_block(idxs[0], idxs[1], jnp.bool_(True),
                      lambda: wait_idx(1))
        @pl.when(blk + 2 < NBLK)
        def _pf0():
            fetch_idx(blk + 2, 0)
        process_block(idxs[1], idxs[0], blk + 2 < NBLK,
                      lambda: wait_idx(0))
        @pl.when(blk + 3 < NBLK)
        def _pf1():
            fetch_idx(blk + 3, 1)
        return carry

    lax.fori_loop(0, NBLK // 2, body, 0)
    plsc.subcore_barrier()

    pltpu.sync_copy(acc.at[rows], out_hbm.at[cid, rows])


_ROWS_BLK = 5120


def _mlp_body(split_out, plo_ref, phi_ref, wa_ref, ba_ref, wb_ref,
              bb_ref, o_ref):
    h = jnp.concatenate([plo_ref[...], phi_ref[...]], axis=-1)
    h = jnp.dot(h, wa_ref[...], preferred_element_type=jnp.float32)
    h = jnp.maximum(h + ba_ref[...], 0.0)
    o = jnp.dot(h, wb_ref[...], preferred_element_type=jnp.float32)
    o = o + bb_ref[...]
    if split_out:
        o = jnp.maximum(o, 0.0)
        o_ref[0] = o[:, :DH]
        o_ref[1] = o[:, DH:]
    else:
        o_ref[...] = o


def _mlp(plo, phi, wa, ba, wb, bb, split_out):
    half_spec = pl.BlockSpec((_ROWS_BLK, DH), lambda i: (i, 0))
    full_spec = pl.BlockSpec((D, D), lambda i: (0, 0))
    bias_spec = pl.BlockSpec((1, D), lambda i: (0, 0))
    if split_out:
        out_spec = pl.BlockSpec((NC, _ROWS_BLK, DH), lambda i: (0, i, 0))
        out_shape = jax.ShapeDtypeStruct((NC, N_PAD, DH), jnp.float32)
    else:
        out_spec = pl.BlockSpec((_ROWS_BLK, D), lambda i: (i, 0))
        out_shape = jax.ShapeDtypeStruct((N_PAD, D), jnp.float32)
    return pl.pallas_call(
        functools.partial(_mlp_body, split_out),
        grid=(N_PAD // _ROWS_BLK,),
        in_specs=[half_spec, half_spec,
                  full_spec, bias_spec, full_spec, bias_spec],
        out_specs=out_spec,
        out_shape=out_shape,
    )(plo, phi, wa, ba.reshape(1, D), wb, bb.reshape(1, D))


def kernel(x, edge_index, W1a, b1a, W1b, b1b, W2a, b2a, W2b, b2b):
    src = edge_index[0].astype(jnp.int32)
    dst = edge_index[1].astype(jnp.int32)
    pad_e = E_PAD - N_EDGES
    src_r = jnp.concatenate([src, jnp.zeros((pad_e,), jnp.int32)])
    src_r = src_r.reshape(NS, NBLK, IB, CSZ)
    dst_r = jnp.concatenate([dst, jnp.full((pad_e,), PAD_DST, jnp.int32)])
    dst_r = dst_r.reshape(NS, NBLK, IB, CSZ)
    idx_comb = jnp.concatenate([src_r, dst_r], axis=2)
    x_pad = jnp.concatenate(
        [x, jnp.zeros((N_PAD - N_NODES, D), jnp.float32)])
    x2 = jnp.stack([x_pad[:, :DH], x_pad[:, DH:]])

    parts1 = _sc_aggregate(x2, idx_comb)
    h1_2 = _mlp(parts1[0], parts1[1], W1a, b1a, W1b, b1b, split_out=True)
    parts2 = _sc_aggregate(h1_2, idx_comb)
    out = _mlp(parts2[0], parts2[1], W2a, b2a, W2b, b2b, split_out=False)
    return out[:N_NODES]
